# baseline (speedup 1.0000x reference)
import functools

import jax
import jax.numpy as jnp
from jax import lax
from jax.experimental import pallas as pl
from jax.experimental.pallas import tpu as pltpu
from jax.experimental.pallas import tpu_sc as plsc

N = 100000
E = 3200000
NC = 2
NS = 16
NW = NC * NS
L = 16
NPAD = 102400
TPN = NPAD // NW
SPN = NPAD // NS
EPT = E // NW
CH = 10000
NCH = EPT // CH

_mesh = lambda: plsc.VectorSubcoreMesh(core_axis_name="c", subcore_axis_name="s")
_params = lambda: pltpu.CompilerParams(needs_layout_passes=False)

_f32 = jnp.float32


def _fill(ref, val, n):
    def body(i, carry):
        ref[pl.ds(i * L, L)] = jnp.full((L,), val, _f32)
        return carry
    lax.fori_loop(0, n // L, body, 0)


def _rsqrt16(d):
    i = plsc.bitcast(d, jnp.int32)
    i = jnp.int32(0x5F3759DF) - (i >> 1)
    r = plsc.bitcast(i, _f32)
    for _ in range(3):
        r = r * (1.5 - 0.5 * d * r * r)
    return r


def _deg_body(src_hbm, dst_hbm, outdeg_hbm, indeg_hbm,
              sidx0, sidx1, sidx2, didx0, didx1, didx2,
              ones_v, zeros_v, acc_out, acc_in,
              ls0, ls1, ss0, ss1, ss2):
    sidx = [sidx0, sidx1, sidx2]
    didx = [didx0, didx1, didx2]
    c = lax.axis_index("c")
    s = lax.axis_index("s")
    _fill(ones_v, 1.0, CH)
    _fill(zeros_v, 0.0, SPN)
    pltpu.sync_copy(zeros_v, acc_out.at[pl.ds(s * SPN, SPN)])
    pltpu.sync_copy(zeros_v, acc_in.at[pl.ds(s * SPN, SPN)])
    plsc.subcore_barrier()
    base = (c * NS + s) * EPT
    LS = [ls0, ls1]
    SS = [ss0, ss1, ss2]
    loads = [None, None]
    scats = [None, None, None]

    def start_load(i):
        b = i % 3
        off = base + i * CH
        loads[i % 2] = (
            pltpu.async_copy(src_hbm.at[pl.ds(off, CH)], sidx[b], LS[i % 2]),
            pltpu.async_copy(dst_hbm.at[pl.ds(off, CH)], didx[b], LS[i % 2]),
        )

    start_load(0)
    for i in range(NCH):
        b = i % 3
        for d in loads[i % 2]:
            d.wait()
        if scats[b] is not None:
            for d in scats[b]:
                d.wait()
            scats[b] = None
        if i + 1 < NCH:
            nb = (i + 1) % 3
            if scats[nb] is not None:
                for d in scats[nb]:
                    d.wait()
                scats[nb] = None
            start_load(i + 1)
        scats[b] = (
            pltpu.async_copy(ones_v, acc_out.at[sidx[b]], SS[b], add=True),
            pltpu.async_copy(ones_v, acc_in.at[didx[b]], SS[b], add=True),
        )
    for b in range(3):
        if scats[b] is not None:
            for d in scats[b]:
                d.wait()
    plsc.subcore_barrier()
    sl = pl.ds(s * SPN, SPN)
    pltpu.sync_copy(acc_out.at[sl], outdeg_hbm.at[c, sl])
    pltpu.sync_copy(acc_in.at[sl], indeg_hbm.at[c, sl])


def _make_deg():
    return functools.partial(
        pl.kernel, _deg_body,
        out_type=(jax.ShapeDtypeStruct((NC, NPAD), _f32),
                  jax.ShapeDtypeStruct((NC, NPAD), _f32)),
        mesh=_mesh(),
        compiler_params=_params(),
        scratch_types=[pltpu.VMEM((CH,), jnp.int32)] * 6 + [
            pltpu.VMEM((CH,), _f32),
            pltpu.VMEM((SPN,), _f32),
            pltpu.VMEM_SHARED((NPAD,), _f32),
            pltpu.VMEM_SHARED((NPAD,), _f32),
        ] + [pltpu.SemaphoreType.DMA] * 5,
        )()


def _norm_body(outdeg_hbm, indeg_hbm, y_hbm,
               ns_hbm, nd_hbm, x_hbm, v_hbm,
               od0, od1, id0, id1, yv, nsv, ndv, xv, vv):
    c = lax.axis_index("c")
    s = lax.axis_index("s")
    w = c * NS + s
    sl = pl.ds(w * TPN, TPN)
    pltpu.sync_copy(outdeg_hbm.at[0, sl], od0)
    pltpu.sync_copy(outdeg_hbm.at[1, sl], od1)
    pltpu.sync_copy(indeg_hbm.at[0, sl], id0)
    pltpu.sync_copy(indeg_hbm.at[1, sl], id1)
    pltpu.sync_copy(y_hbm.at[sl], yv)

    def body(i, carry):
        d = pl.ds(i * L, L)
        do = od0[d] + od1[d]
        di = id0[d] + id1[d]
        ns16 = jnp.where(do > 0.0, _rsqrt16(jnp.maximum(do, 1.0)), 0.0)
        nd16 = jnp.where(di > 0.0, _rsqrt16(jnp.maximum(di, 1.0)), 0.0)
        y16 = yv[d]
        x16 = jnp.where(y16 == 0.0, -1.0, y16)
        nsv[d] = ns16
        ndv[d] = nd16
        xv[d] = x16
        vv[d] = x16 * ns16
        return carry

    lax.fori_loop(0, TPN // L, body, 0)
    pltpu.sync_copy(nsv, ns_hbm.at[sl])
    pltpu.sync_copy(ndv, nd_hbm.at[sl])
    pltpu.sync_copy(xv, x_hbm.at[sl])
    pltpu.sync_copy(vv, v_hbm.at[sl])


def _make_norm():
    vecs = jax.ShapeDtypeStruct((NPAD,), _f32)
    return functools.partial(
        pl.kernel, _norm_body,
        out_type=(vecs,) * 4,
        mesh=_mesh(),
        compiler_params=_params(),
        scratch_types=[pltpu.VMEM((TPN,), _f32)] * 9,
    )()


SCH = 2000
SNCH = EPT // SCH
ZN = 1600


def _spmv_body(v_hbm, src_hbm, dst_hbm, accp_hbm,
               v_loc, sidx0, sidx1, didx0, didx1, didx2, didx3,
               vals0, vals1, vals2, vals3, zeros_v,
               acc_sh, vsem, ls0, ls1, ss0, ss1, ss2, ss3):
    sidx = [sidx0, sidx1]
    didx = [didx0, didx1, didx2, didx3]
    vals = [vals0, vals1, vals2, vals3]
    c = lax.axis_index("c")
    s = lax.axis_index("s")
    vload = pltpu.async_copy(v_hbm.at[pl.ds(0, N)], v_loc, vsem)
    _fill(zeros_v, 0.0, ZN)
    sl = pl.ds(s * SPN, SPN)
    for z in range(SPN // ZN):
        pltpu.sync_copy(zeros_v, acc_sh.at[pl.ds(s * SPN + z * ZN, ZN)])
    base = (c * NS + s) * EPT
    LS = [ls0, ls1]
    SS = [ss0, ss1, ss2, ss3]
    loads = [None, None]
    scats = [None, None, None, None]

    def start_load(i):
        off = base + i * SCH
        loads[i % 2] = (
            pltpu.async_copy(src_hbm.at[pl.ds(off, SCH)], sidx[i % 2],
                             LS[i % 2]),
            pltpu.async_copy(dst_hbm.at[pl.ds(off, SCH)], didx[i % 4],
                             LS[i % 2]),
        )

    def gather(i):
        sb = sidx[i % 2]
        vb = vals[i % 4]

        @plsc.parallel_loop(0, SCH // L, unroll=8)
        def gather_loop(j):
            d = pl.ds(j * L, L)
            vb[d] = plsc.load_gather(v_loc, [sb[d]])

    start_load(0)
    start_load(1)
    vload.wait()
    plsc.subcore_barrier()
    for d in loads[0]:
        d.wait()
    gather(0)
    for i in range(SNCH):
        bd = i % 4
        if scats[bd] is not None:
            scats[bd].wait()
            scats[bd] = None
        scats[bd] = pltpu.async_copy(vals[bd], acc_sh.at[didx[bd]],
                                     SS[bd], add=True)
        if i + 1 < SNCH:
            for d in loads[(i + 1) % 2]:
                d.wait()
            nb = (i + 1) % 4
            if scats[nb] is not None:
                scats[nb].wait()
                scats[nb] = None
            gather(i + 1)
            if i + 2 < SNCH:
                pb = (i + 2) % 4
                if scats[pb] is not None:
                    scats[pb].wait()
                    scats[pb] = None
                start_load(i + 2)
    for b in range(4):
        if scats[b] is not None:
            scats[b].wait()
    plsc.subcore_barrier()
    pltpu.sync_copy(acc_sh.at[sl], accp_hbm.at[c, sl])


def _make_spmv():
    return functools.partial(
        pl.kernel, _spmv_body,
        out_type=jax.ShapeDtypeStruct((NC, NPAD), _f32),
        mesh=_mesh(),
        compiler_params=_params(),
        scratch_types=[
            pltpu.VMEM((N,), _f32),
            pltpu.VMEM((SCH,), jnp.int32),
            pltpu.VMEM((SCH,), jnp.int32),
            pltpu.VMEM((SCH,), jnp.int32),
            pltpu.VMEM((SCH,), jnp.int32),
            pltpu.VMEM((SCH,), jnp.int32),
            pltpu.VMEM((SCH,), jnp.int32),
            pltpu.VMEM((SCH,), _f32),
            pltpu.VMEM((SCH,), _f32),
            pltpu.VMEM((SCH,), _f32),
            pltpu.VMEM((SCH,), _f32),
            pltpu.VMEM((ZN,), _f32),
            pltpu.VMEM_SHARED((NPAD,), _f32),
        ] + [pltpu.SemaphoreType.DMA] * 7,
        )()


def _nodeA_body(accp_hbm, nd_hbm, ns_hbm, x_hbm, consts_hbm, v_hbm,
                a0, a1v, ndv, nsv, xv, vv, cv):
    c = lax.axis_index("c")
    s = lax.axis_index("s")
    w = c * NS + s
    sl = pl.ds(w * TPN, TPN)
    pltpu.sync_copy(accp_hbm.at[0, sl], a0)
    pltpu.sync_copy(accp_hbm.at[1, sl], a1v)
    pltpu.sync_copy(nd_hbm.at[sl], ndv)
    pltpu.sync_copy(ns_hbm.at[sl], nsv)
    pltpu.sync_copy(x_hbm.at[sl], xv)
    pltpu.sync_copy(consts_hbm, cv)

    def body(i, carry):
        d = pl.ds(i * L, L)
        a = (a0[d] + a1v[d]) * ndv[d]
        x = xv[d]
        var = (cv[40] * a * a + cv[41] * x * x + cv[42] * a * x
               + cv[43] * a + cv[44] * x + cv[45])
        inv = _rsqrt16(var)
        ai = a * inv
        xi = x * inv
        alpha = cv[46]
        q = jnp.zeros((L,), _f32)
        for k in range(8):
            t = ai * cv[k] + xi * cv[8 + k] + inv * cv[16 + k] + cv[24 + k]
            p = jnp.maximum(t, 0.0) + alpha * jnp.minimum(t, 0.0)
            q = q + p * cv[32 + k]
        vv[d] = q * nsv[d]
        return carry

    lax.fori_loop(0, TPN // L, body, 0)
    pltpu.sync_copy(vv, v_hbm.at[sl])


def _make_nodeA():
    return functools.partial(
        pl.kernel, _nodeA_body,
        out_type=jax.ShapeDtypeStruct((NPAD,), _f32),
        mesh=_mesh(),
        compiler_params=_params(),
        scratch_types=[pltpu.VMEM((TPN,), _f32)] * 6
        + [pltpu.VMEM((48, L), _f32)],
    )()


def _nodeB_body(accp_hbm, nd_hbm, ns_hbm, x_hbm, consts_hbm,
                xn_hbm, vn_hbm,
                a0, a1v, ndv, nsv, xv, xnv, vnv, cv):
    c = lax.axis_index("c")
    s = lax.axis_index("s")
    w = c * NS + s
    sl = pl.ds(w * TPN, TPN)
    pltpu.sync_copy(accp_hbm.at[0, sl], a0)
    pltpu.sync_copy(accp_hbm.at[1, sl], a1v)
    pltpu.sync_copy(nd_hbm.at[sl], ndv)
    pltpu.sync_copy(ns_hbm.at[sl], nsv)
    pltpu.sync_copy(x_hbm.at[sl], xv)
    pltpu.sync_copy(consts_hbm, cv)
    b2v = cv[0]

    def body(i, carry):
        d = pl.ds(i * L, L)
        xn = xv[d] + (a0[d] + a1v[d]) * ndv[d] + b2v
        xnv[d] = xn
        vnv[d] = xn * nsv[d]
        return carry

    lax.fori_loop(0, TPN // L, body, 0)
    pltpu.sync_copy(xnv, xn_hbm.at[sl])
    pltpu.sync_copy(vnv, vn_hbm.at[sl])


def _make_nodeB():
    vecs = jax.ShapeDtypeStruct((NPAD,), _f32)
    return functools.partial(
        pl.kernel, _nodeB_body,
        out_type=(vecs, vecs),
        mesh=_mesh(),
        compiler_params=_params(),
        scratch_types=[pltpu.VMEM((TPN,), _f32)] * 7
        + [pltpu.VMEM((1, L), _f32)],
    )()


def kernel(y, edge_index, W1, b1, res_W1, a1, ln_g, ln_b, W2, b2):
    src = edge_index[0]
    dst = edge_index[1]
    ypad = jnp.pad(y[:, 0], (0, NPAD - N))

    u = W1[0]
    r = res_W1[0]
    du = u - u.mean()
    dr = r - r.mean()
    dc = b1 - b1.mean()
    scal = jnp.stack([
        (du * du).mean(), (dr * dr).mean(), 2.0 * (du * dr).mean(),
        2.0 * (du * dc).mean(), 2.0 * (dr * dc).mean(),
        (dc * dc).mean() + 1e-5, a1[0], jnp.float32(0.0),
    ])
    chan = jnp.concatenate([du * ln_g, dr * ln_g, dc * ln_g, ln_b,
                            W2[:, 0], scal]).astype(_f32)
    rows = jnp.broadcast_to(chan[:, None], (48, L))
    b2row = jnp.broadcast_to(b2[0], (1, L)).astype(_f32)

    deg = _make_deg()
    norm = _make_norm()
    spmv = _make_spmv()
    nodeA = _make_nodeA()
    nodeB = _make_nodeB()

    outdeg, indeg = deg(src, dst)
    ns, nd, x0, v1 = norm(outdeg, indeg, ypad)
    s1 = spmv(v1, src, dst)
    v2 = nodeA(s1, nd, ns, x0, rows)
    s2 = spmv(v2, src, dst)
    x1, v3 = nodeB(s2, nd, ns, x0, b2row)
    s3 = spmv(v3, src, dst)
    v4 = nodeA(s3, nd, ns, x1, rows)
    s4 = spmv(v4, src, dst)
    x2, _ = nodeB(s4, nd, ns, x1, b2row)
    return x2[:N].reshape(N, 1)


# speedup vs baseline: 1.0555x; 1.0555x over previous
import functools

import jax
import jax.numpy as jnp
from jax import lax
from jax.experimental import pallas as pl
from jax.experimental.pallas import tpu as pltpu
from jax.experimental.pallas import tpu_sc as plsc

N = 100000
E = 3200000
NC = 2
NS = 16
NW = NC * NS
L = 16
NPAD = 102400
TPN = NPAD // NW
SPN = NPAD // NS
EPT = E // NW
CH = 10000
NCH = EPT // CH

_mesh = lambda: plsc.VectorSubcoreMesh(core_axis_name="c", subcore_axis_name="s")
_params = lambda: pltpu.CompilerParams(needs_layout_passes=False)

_f32 = jnp.float32


def _fill(ref, val, n):
    def body(i, carry):
        ref[pl.ds(i * L, L)] = jnp.full((L,), val, _f32)
        return carry
    lax.fori_loop(0, n // L, body, 0)


def _rsqrt16(d):
    i = plsc.bitcast(d, jnp.int32)
    i = jnp.int32(0x5F3759DF) - (i >> 1)
    r = plsc.bitcast(i, _f32)
    for _ in range(3):
        r = r * (1.5 - 0.5 * d * r * r)
    return r


def _deg_body(src_hbm, dst_hbm, outdeg_hbm, indeg_hbm,
              sidx0, sidx1, sidx2, didx0, didx1, didx2,
              ones_v, zeros_v, acc_out, acc_in,
              ls0, ls1, ss0, ss1, ss2):
    sidx = [sidx0, sidx1, sidx2]
    didx = [didx0, didx1, didx2]
    c = lax.axis_index("c")
    s = lax.axis_index("s")
    _fill(ones_v, 1.0, CH)
    _fill(zeros_v, 0.0, SPN)
    pltpu.sync_copy(zeros_v, acc_out.at[pl.ds(s * SPN, SPN)])
    pltpu.sync_copy(zeros_v, acc_in.at[pl.ds(s * SPN, SPN)])
    plsc.subcore_barrier()
    base = (c * NS + s) * EPT
    LS = [ls0, ls1]
    SS = [ss0, ss1, ss2]
    loads = [None, None]
    scats = [None, None, None]

    def start_load(i):
        b = i % 3
        off = base + i * CH
        loads[i % 2] = (
            pltpu.async_copy(src_hbm.at[pl.ds(off, CH)], sidx[b], LS[i % 2]),
            pltpu.async_copy(dst_hbm.at[pl.ds(off, CH)], didx[b], LS[i % 2]),
        )

    start_load(0)
    for i in range(NCH):
        b = i % 3
        for d in loads[i % 2]:
            d.wait()
        if scats[b] is not None:
            for d in scats[b]:
                d.wait()
            scats[b] = None
        if i + 1 < NCH:
            nb = (i + 1) % 3
            if scats[nb] is not None:
                for d in scats[nb]:
                    d.wait()
                scats[nb] = None
            start_load(i + 1)
        scats[b] = (
            pltpu.async_copy(ones_v, acc_out.at[sidx[b]], SS[b], add=True),
            pltpu.async_copy(ones_v, acc_in.at[didx[b]], SS[b], add=True),
        )
    for b in range(3):
        if scats[b] is not None:
            for d in scats[b]:
                d.wait()
    plsc.subcore_barrier()
    sl = pl.ds(s * SPN, SPN)
    pltpu.sync_copy(acc_out.at[sl], outdeg_hbm.at[c, sl])
    pltpu.sync_copy(acc_in.at[sl], indeg_hbm.at[c, sl])


def _make_deg():
    return functools.partial(
        pl.kernel, _deg_body,
        out_type=(jax.ShapeDtypeStruct((NC, NPAD), _f32),
                  jax.ShapeDtypeStruct((NC, NPAD), _f32)),
        mesh=_mesh(),
        compiler_params=_params(),
        scratch_types=[pltpu.VMEM((CH,), jnp.int32)] * 6 + [
            pltpu.VMEM((CH,), _f32),
            pltpu.VMEM((SPN,), _f32),
            pltpu.VMEM_SHARED((NPAD,), _f32),
            pltpu.VMEM_SHARED((NPAD,), _f32),
        ] + [pltpu.SemaphoreType.DMA] * 5,
        )()


def _norm_body(outdeg_hbm, indeg_hbm, y_hbm,
               ns_hbm, nd_hbm, x_hbm, v_hbm,
               od0, od1, id0, id1, yv, nsv, ndv, xv, vv):
    c = lax.axis_index("c")
    s = lax.axis_index("s")
    w = c * NS + s
    sl = pl.ds(w * TPN, TPN)
    pltpu.sync_copy(outdeg_hbm.at[0, sl], od0)
    pltpu.sync_copy(outdeg_hbm.at[1, sl], od1)
    pltpu.sync_copy(indeg_hbm.at[0, sl], id0)
    pltpu.sync_copy(indeg_hbm.at[1, sl], id1)
    pltpu.sync_copy(y_hbm.at[sl], yv)

    @plsc.parallel_loop(0, TPN // L, unroll=2)
    def body(i):
        d = pl.ds(i * L, L)
        do = od0[d] + od1[d]
        di = id0[d] + id1[d]
        ns16 = jnp.where(do > 0.0, _rsqrt16(jnp.maximum(do, 1.0)), 0.0)
        nd16 = jnp.where(di > 0.0, _rsqrt16(jnp.maximum(di, 1.0)), 0.0)
        y16 = yv[d]
        x16 = jnp.where(y16 == 0.0, -1.0, y16)
        nsv[d] = ns16
        ndv[d] = nd16
        xv[d] = x16
        vv[d] = x16 * ns16
    pltpu.sync_copy(nsv, ns_hbm.at[sl])
    pltpu.sync_copy(ndv, nd_hbm.at[sl])
    pltpu.sync_copy(xv, x_hbm.at[sl])
    pltpu.sync_copy(vv, v_hbm.at[sl])


def _make_norm():
    vecs = jax.ShapeDtypeStruct((NPAD,), _f32)
    return functools.partial(
        pl.kernel, _norm_body,
        out_type=(vecs,) * 4,
        mesh=_mesh(),
        compiler_params=_params(),
        scratch_types=[pltpu.VMEM((TPN,), _f32)] * 9,
    )()


ZN = 1600


def _spmv_body(v_hbm, src_hbm, dst_hbm, accp_hbm,
               sidx0, sidx1, didx0, didx1, didx2,
               vals0, vals1, vals2, zeros_v,
               v_sh, acc_sh, ls0, ls1, gs0, gs1, gs2, ss0, ss1, ss2):
    sidx = [sidx0, sidx1]
    didx = [didx0, didx1, didx2]
    vals = [vals0, vals1, vals2]
    c = lax.axis_index("c")
    s = lax.axis_index("s")
    _fill(zeros_v, 0.0, ZN)
    sl = pl.ds(s * SPN, SPN)
    pltpu.sync_copy(v_hbm.at[sl], v_sh.at[sl])
    for z in range(SPN // ZN):
        pltpu.sync_copy(zeros_v, acc_sh.at[pl.ds(s * SPN + z * ZN, ZN)])
    base = (c * NS + s) * EPT
    LS = [ls0, ls1]
    GS = [gs0, gs1, gs2]
    SS = [ss0, ss1, ss2]
    loads = [None, None]
    gaths = [None, None, None]
    scats = [None, None, None]

    def start_load(i):
        off = base + i * CH
        loads[i % 2] = (
            pltpu.async_copy(src_hbm.at[pl.ds(off, CH)], sidx[i % 2],
                             LS[i % 2]),
            pltpu.async_copy(dst_hbm.at[pl.ds(off, CH)], didx[i % 3],
                             LS[i % 2]),
        )

    start_load(0)
    plsc.subcore_barrier()
    for i in range(NCH):
        bs = i % 2
        bd = i % 3
        for d in loads[bs]:
            d.wait()
        if scats[bd] is not None:
            scats[bd].wait()
            scats[bd] = None
        gaths[bd] = pltpu.async_copy(v_sh.at[sidx[bs]], vals[bd], GS[bd])
        pb = (i - 1) % 3
        if i > 0:
            gaths[pb].wait()
            gaths[pb] = None
            scats[pb] = pltpu.async_copy(vals[pb], acc_sh.at[didx[pb]],
                                         SS[pb], add=True)
        if i + 1 < NCH:
            nb = (i + 1) % 3
            if scats[nb] is not None:
                scats[nb].wait()
                scats[nb] = None
            start_load(i + 1)
    lb = (NCH - 1) % 3
    gaths[lb].wait()
    scats[lb] = pltpu.async_copy(vals[lb], acc_sh.at[didx[lb]],
                                 SS[lb], add=True)
    for b in range(3):
        if scats[b] is not None:
            scats[b].wait()
    plsc.subcore_barrier()
    pltpu.sync_copy(acc_sh.at[sl], accp_hbm.at[c, sl])


def _make_spmv():
    return functools.partial(
        pl.kernel, _spmv_body,
        out_type=jax.ShapeDtypeStruct((NC, NPAD), _f32),
        mesh=_mesh(),
        compiler_params=_params(),
        scratch_types=[
            pltpu.VMEM((CH,), jnp.int32),
            pltpu.VMEM((CH,), jnp.int32),
            pltpu.VMEM((CH,), jnp.int32),
            pltpu.VMEM((CH,), jnp.int32),
            pltpu.VMEM((CH,), jnp.int32),
            pltpu.VMEM((CH,), _f32),
            pltpu.VMEM((CH,), _f32),
            pltpu.VMEM((CH,), _f32),
            pltpu.VMEM((ZN,), _f32),
            pltpu.VMEM_SHARED((NPAD,), _f32),
            pltpu.VMEM_SHARED((NPAD,), _f32),
        ] + [pltpu.SemaphoreType.DMA] * 8,
        )()


def _nodeA_body(accp_hbm, nd_hbm, ns_hbm, x_hbm, consts_hbm, v_hbm,
                a0, a1v, ndv, nsv, xv, vv, cv):
    c = lax.axis_index("c")
    s = lax.axis_index("s")
    w = c * NS + s
    sl = pl.ds(w * TPN, TPN)
    pltpu.sync_copy(accp_hbm.at[0, sl], a0)
    pltpu.sync_copy(accp_hbm.at[1, sl], a1v)
    pltpu.sync_copy(nd_hbm.at[sl], ndv)
    pltpu.sync_copy(ns_hbm.at[sl], nsv)
    pltpu.sync_copy(x_hbm.at[sl], xv)
    pltpu.sync_copy(consts_hbm, cv)
    P = [cv[k] for k in range(8)]
    Q = [cv[8 + k] for k in range(8)]
    R = [cv[16 + k] for k in range(8)]
    S = [cv[24 + k] for k in range(8)]
    W2 = [cv[32 + k] for k in range(8)]
    A2 = cv[40]
    B2 = cv[41]
    C2 = cv[42]
    D2 = cv[43]
    E2 = cv[44]
    F2 = cv[45]
    alpha = cv[46]

    @plsc.parallel_loop(0, TPN // L, unroll=2)
    def body(i):
        d = pl.ds(i * L, L)
        a = (a0[d] + a1v[d]) * ndv[d]
        x = xv[d]
        var = (A2 * a + C2 * x + D2) * a + (B2 * x + E2) * x + F2
        inv = _rsqrt16(var)
        ai = a * inv
        xi = x * inv
        q0 = jnp.zeros((L,), _f32)
        q1 = jnp.zeros((L,), _f32)
        for k in range(0, 8, 2):
            t = ai * P[k] + xi * Q[k] + (inv * R[k] + S[k])
            u = ai * P[k + 1] + xi * Q[k + 1] + (inv * R[k + 1] + S[k + 1])
            q0 = q0 + (jnp.maximum(t, 0.0) + alpha * jnp.minimum(t, 0.0)) * W2[k]
            q1 = q1 + (jnp.maximum(u, 0.0) + alpha * jnp.minimum(u, 0.0)) * W2[k + 1]
        vv[d] = (q0 + q1) * nsv[d]

    pltpu.sync_copy(vv, v_hbm.at[sl])


def _make_nodeA():
    return functools.partial(
        pl.kernel, _nodeA_body,
        out_type=jax.ShapeDtypeStruct((NPAD,), _f32),
        mesh=_mesh(),
        compiler_params=_params(),
        scratch_types=[pltpu.VMEM((TPN,), _f32)] * 6
        + [pltpu.VMEM((48, L), _f32)],
    )()


def _nodeB_body(accp_hbm, nd_hbm, ns_hbm, x_hbm, consts_hbm,
                xn_hbm, vn_hbm,
                a0, a1v, ndv, nsv, xv, xnv, vnv, cv):
    c = lax.axis_index("c")
    s = lax.axis_index("s")
    w = c * NS + s
    sl = pl.ds(w * TPN, TPN)
    pltpu.sync_copy(accp_hbm.at[0, sl], a0)
    pltpu.sync_copy(accp_hbm.at[1, sl], a1v)
    pltpu.sync_copy(nd_hbm.at[sl], ndv)
    pltpu.sync_copy(ns_hbm.at[sl], nsv)
    pltpu.sync_copy(x_hbm.at[sl], xv)
    pltpu.sync_copy(consts_hbm, cv)
    b2v = cv[0]
    @plsc.parallel_loop(0, TPN // L, unroll=4)
    def body(i):
        d = pl.ds(i * L, L)
        xn = xv[d] + (a0[d] + a1v[d]) * ndv[d] + b2v
        xnv[d] = xn
        vnv[d] = xn * nsv[d]
    pltpu.sync_copy(xnv, xn_hbm.at[sl])
    pltpu.sync_copy(vnv, vn_hbm.at[sl])


def _make_nodeB():
    vecs = jax.ShapeDtypeStruct((NPAD,), _f32)
    return functools.partial(
        pl.kernel, _nodeB_body,
        out_type=(vecs, vecs),
        mesh=_mesh(),
        compiler_params=_params(),
        scratch_types=[pltpu.VMEM((TPN,), _f32)] * 7
        + [pltpu.VMEM((1, L), _f32)],
    )()


def kernel(y, edge_index, W1, b1, res_W1, a1, ln_g, ln_b, W2, b2):
    src = edge_index[0]
    dst = edge_index[1]
    ypad = jnp.pad(y[:, 0], (0, NPAD - N))

    u = W1[0]
    r = res_W1[0]
    du = u - u.mean()
    dr = r - r.mean()
    dc = b1 - b1.mean()
    scal = jnp.stack([
        (du * du).mean(), (dr * dr).mean(), 2.0 * (du * dr).mean(),
        2.0 * (du * dc).mean(), 2.0 * (dr * dc).mean(),
        (dc * dc).mean() + 1e-5, a1[0], jnp.float32(0.0),
    ])
    chan = jnp.concatenate([du * ln_g, dr * ln_g, dc * ln_g, ln_b,
                            W2[:, 0], scal]).astype(_f32)
    rows = jnp.broadcast_to(chan[:, None], (48, L))
    b2row = jnp.broadcast_to(b2[0], (1, L)).astype(_f32)

    deg = _make_deg()
    norm = _make_norm()
    spmv = _make_spmv()
    nodeA = _make_nodeA()
    nodeB = _make_nodeB()

    outdeg, indeg = deg(src, dst)
    ns, nd, x0, v1 = norm(outdeg, indeg, ypad)
    s1 = spmv(v1, src, dst)
    v2 = nodeA(s1, nd, ns, x0, rows)
    s2 = spmv(v2, src, dst)
    x1, v3 = nodeB(s2, nd, ns, x0, b2row)
    s3 = spmv(v3, src, dst)
    v4 = nodeA(s3, nd, ns, x1, rows)
    s4 = spmv(v4, src, dst)
    x2, _ = nodeB(s4, nd, ns, x1, b2row)
    return x2[:N].reshape(N, 1)
